# baseline (speedup 1.0000x reference)
import jax
import jax.numpy as jnp
from jax.experimental import pallas as pl
from jax.experimental.pallas import tpu as pltpu

B_TILE = 256
KA_PER = 8


def _zdec_kernel(phi_ref, x0a_ref, x0b_ref, x1_ref, w1phiT_ref,
                 e0lo_ref, e0hi_ref, e12_ref, dcat2_ref,
                 w2Tq_ref, b2q_ref, w3lo_ref, w3hi_ref, b3o_ref,
                 out_ref, phi4b_ref):
    j = pl.program_id(1)

    @pl.when(j == 0)
    def _build_phi_scratch():
        phiW = jnp.dot(phi_ref[...], w1phiT_ref[...],
                       preferred_element_type=jnp.float32)
        phi4 = jnp.concatenate([phiW, phiW, phiW, phiW],
                               axis=-1).astype(jnp.bfloat16)
        phi4b_ref[...] = jnp.broadcast_to(phi4[:, None, :],
                                          (B_TILE, 16, 256))

    phi4b = phi4b_ref[...]
    for t in range(KA_PER):
        s1 = x1_ref[0, j * KA_PER + t]
        cc2 = (x0a_ref[...] * e0lo_ref[...] + x0b_ref[...] * e0hi_ref[...]
               + s1 * e12_ref[...] + dcat2_ref[...]).astype(jnp.bfloat16)
        pre = phi4b + cc2[None, :, :]
        h1 = jnp.maximum(pre, jnp.bfloat16(0)).reshape(B_TILE * 16, 256)
        a2 = jnp.dot(h1, w2Tq_ref[...],
                     preferred_element_type=jnp.float32).astype(jnp.bfloat16)
        h2 = (jnp.maximum(a2 + b2q_ref[...], jnp.bfloat16(0))
              .reshape(B_TILE, 16, 256))
        h2lo = h2[:, :8, :].reshape(B_TILE * 8, 256)
        h2hi = h2[:, 8:, :].reshape(B_TILE * 8, 256)
        o = (jnp.dot(h2lo, w3lo_ref[...], preferred_element_type=jnp.float32)
             + jnp.dot(h2hi, w3hi_ref[...], preferred_element_type=jnp.float32)
             + b3o_ref[...])
        out_ref[:, t * 8:(t + 1) * 8, :] = o.reshape(B_TILE, 8, 128)


def kernel(phi, region_params, W1, b1, W2, b2, W3, b3):
    B, PHI = phi.shape
    levels, R, _ = region_params.shape
    H = W2.shape[0]
    O = W3.shape[0]
    K = R ** levels

    x0 = region_params[0, :, 0]
    row_k0 = jnp.concatenate([jnp.arange(0, R, 4), jnp.arange(2, R, 4)])
    x0a = x0[row_k0].reshape(R // 2, 1)
    x0b = x0[row_k0 + 1].reshape(R // 2, 1)
    x1 = region_params[1, :, 0].reshape(1, R)
    w1phiT = W1[:, 2:2 + PHI].T
    e0 = jnp.concatenate([W1[:, 0], W1[:, 1]])
    e1 = jnp.concatenate([W1[:, 1], W1[:, 0]])
    dcat = jnp.concatenate([b1 + W1[:, 2 + PHI], b1 + W1[:, 3 + PHI]])
    z128 = jnp.zeros((2 * H,), jnp.float32)
    e0lo = jnp.concatenate([e0, z128]).reshape(1, 4 * H)
    e0hi = jnp.concatenate([z128, e0]).reshape(1, 4 * H)
    e12 = jnp.concatenate([e1, e1]).reshape(1, 4 * H)
    dcat2 = jnp.concatenate([dcat, dcat]).reshape(1, 4 * H)

    w2T = W2.T
    w3T = W3.T
    Z2 = jnp.zeros((H, H), jnp.float32)
    Z3 = jnp.zeros((H, O), jnp.float32)

    def blkdiag4(M, Zm):
        r1 = jnp.concatenate([M, Zm, Zm, Zm], axis=1)
        r2 = jnp.concatenate([Zm, M, Zm, Zm], axis=1)
        r3 = jnp.concatenate([Zm, Zm, M, Zm], axis=1)
        r4 = jnp.concatenate([Zm, Zm, Zm, M], axis=1)
        return jnp.concatenate([r1, r2, r3, r4], axis=0)

    w2Tq = blkdiag4(w2T, Z2).astype(jnp.bfloat16)
    w3Tq = blkdiag4(w3T, Z3)
    z64 = jnp.zeros((4 * H, 4 * O), jnp.float32)
    w3lo = jnp.concatenate([w3Tq, z64], axis=1).astype(jnp.bfloat16)
    w3hi = jnp.concatenate([z64, w3Tq], axis=1).astype(jnp.bfloat16)
    b2q = jnp.concatenate([b2, b2, b2, b2]).reshape(1, 4 * H).astype(jnp.bfloat16)
    b3o = jnp.concatenate([b3] * 8).reshape(1, 8 * O)

    grid = (B // B_TILE, R // KA_PER)
    out = pl.pallas_call(
        _zdec_kernel,
        grid=grid,
        in_specs=[
            pl.BlockSpec((B_TILE, PHI), lambda i, j: (i, 0)),
            pl.BlockSpec((R // 2, 1), lambda i, j: (0, 0)),
            pl.BlockSpec((R // 2, 1), lambda i, j: (0, 0)),
            pl.BlockSpec(memory_space=pltpu.SMEM),
            pl.BlockSpec((PHI, H), lambda i, j: (0, 0)),
            pl.BlockSpec((1, 4 * H), lambda i, j: (0, 0)),
            pl.BlockSpec((1, 4 * H), lambda i, j: (0, 0)),
            pl.BlockSpec((1, 4 * H), lambda i, j: (0, 0)),
            pl.BlockSpec((1, 4 * H), lambda i, j: (0, 0)),
            pl.BlockSpec((4 * H, 4 * H), lambda i, j: (0, 0)),
            pl.BlockSpec((1, 4 * H), lambda i, j: (0, 0)),
            pl.BlockSpec((4 * H, 8 * O), lambda i, j: (0, 0)),
            pl.BlockSpec((4 * H, 8 * O), lambda i, j: (0, 0)),
            pl.BlockSpec((1, 8 * O), lambda i, j: (0, 0)),
        ],
        out_specs=pl.BlockSpec((B_TILE, KA_PER * 8, 8 * O),
                               lambda i, j: (i, j, 0)),
        out_shape=jax.ShapeDtypeStruct((B, K // 4, 8 * O), jnp.float32),
        scratch_shapes=[pltpu.VMEM((B_TILE, 16, 256), jnp.bfloat16)],
        compiler_params=pltpu.CompilerParams(
            dimension_semantics=("parallel", "arbitrary")),
        interpret=False,
    )(phi, x0a, x0b, x1, w1phiT, e0lo, e0hi, e12, dcat2, w2Tq, b2q,
      w3lo, w3hi, b3o)
    return out.reshape(B, K, 2 * O)


# speedup vs baseline: 1.0220x; 1.0220x over previous
import jax
import jax.numpy as jnp
from jax.experimental import pallas as pl
from jax.experimental.pallas import tpu as pltpu

B_TILE = 256
KA_PER = 8


def _zdec_kernel(phi_ref, x0a_ref, x0b_ref, x1_ref, w1phiT_ref,
                 e0lo_ref, e0hi_ref, e12_ref, dcat2_ref,
                 w2Tq_ref, b2q_ref, w3Tq_ref, b3q_ref, out_ref, phi4b_ref):
    j = pl.program_id(1)

    @pl.when(j == 0)
    def _build_phi_scratch():
        phiW = jnp.dot(phi_ref[...], w1phiT_ref[...],
                       preferred_element_type=jnp.float32)
        phi4 = jnp.concatenate([phiW, phiW, phiW, phiW],
                               axis=-1).astype(jnp.bfloat16)
        phi4b_ref[...] = jnp.broadcast_to(phi4[:, None, :],
                                          (B_TILE, 16, 256))

    phi4b = phi4b_ref[...]
    for t in range(KA_PER):
        s1 = x1_ref[0, j * KA_PER + t]
        cc2 = (x0a_ref[...] * e0lo_ref[...] + x0b_ref[...] * e0hi_ref[...]
               + s1 * e12_ref[...] + dcat2_ref[...]).astype(jnp.bfloat16)
        pre = phi4b + cc2[None, :, :]
        h1 = jnp.maximum(pre, jnp.bfloat16(0)).reshape(B_TILE * 16, 256)
        a2 = jnp.dot(h1, w2Tq_ref[...],
                     preferred_element_type=jnp.float32).astype(jnp.bfloat16)
        h2 = jnp.maximum(a2 + b2q_ref[...], jnp.bfloat16(0))
        o = (jnp.dot(h2, w3Tq_ref[...], preferred_element_type=jnp.float32)
             + b3q_ref[...]).reshape(B_TILE, 16, 64)
        out_ref[:, t * 8:(t + 1) * 8, :] = jnp.concatenate(
            [o[:, :8, :], o[:, 8:, :]], axis=-1)


def kernel(phi, region_params, W1, b1, W2, b2, W3, b3):
    B, PHI = phi.shape
    levels, R, _ = region_params.shape
    H = W2.shape[0]
    O = W3.shape[0]
    K = R ** levels

    x0 = region_params[0, :, 0]
    row_k0 = jnp.concatenate([jnp.arange(0, R, 4), jnp.arange(2, R, 4)])
    x0a = x0[row_k0].reshape(R // 2, 1)
    x0b = x0[row_k0 + 1].reshape(R // 2, 1)
    x1 = region_params[1, :, 0].reshape(1, R)
    w1phiT = W1[:, 2:2 + PHI].T
    e0 = jnp.concatenate([W1[:, 0], W1[:, 1]])
    e1 = jnp.concatenate([W1[:, 1], W1[:, 0]])
    dcat = jnp.concatenate([b1 + W1[:, 2 + PHI], b1 + W1[:, 3 + PHI]])
    z128 = jnp.zeros((2 * H,), jnp.float32)
    e0lo = jnp.concatenate([e0, z128]).reshape(1, 4 * H)
    e0hi = jnp.concatenate([z128, e0]).reshape(1, 4 * H)
    e12 = jnp.concatenate([e1, e1]).reshape(1, 4 * H)
    dcat2 = jnp.concatenate([dcat, dcat]).reshape(1, 4 * H)

    w2T = W2.T
    w3T = W3.T
    Z2 = jnp.zeros((H, H), jnp.float32)
    Z3 = jnp.zeros((H, O), jnp.float32)

    def blkdiag4(M, Zm):
        r1 = jnp.concatenate([M, Zm, Zm, Zm], axis=1)
        r2 = jnp.concatenate([Zm, M, Zm, Zm], axis=1)
        r3 = jnp.concatenate([Zm, Zm, M, Zm], axis=1)
        r4 = jnp.concatenate([Zm, Zm, Zm, M], axis=1)
        return jnp.concatenate([r1, r2, r3, r4], axis=0)

    w2Tq = blkdiag4(w2T, Z2).astype(jnp.bfloat16)
    w3Tq = blkdiag4(w3T, Z3).astype(jnp.bfloat16)
    b2q = jnp.concatenate([b2, b2, b2, b2]).reshape(1, 4 * H).astype(jnp.bfloat16)
    b3q = jnp.concatenate([b3, b3, b3, b3]).reshape(1, 4 * O)

    grid = (B // B_TILE, R // KA_PER)
    out = pl.pallas_call(
        _zdec_kernel,
        grid=grid,
        in_specs=[
            pl.BlockSpec((B_TILE, PHI), lambda i, j: (i, 0)),
            pl.BlockSpec((R // 2, 1), lambda i, j: (0, 0)),
            pl.BlockSpec((R // 2, 1), lambda i, j: (0, 0)),
            pl.BlockSpec(memory_space=pltpu.SMEM),
            pl.BlockSpec((PHI, H), lambda i, j: (0, 0)),
            pl.BlockSpec((1, 4 * H), lambda i, j: (0, 0)),
            pl.BlockSpec((1, 4 * H), lambda i, j: (0, 0)),
            pl.BlockSpec((1, 4 * H), lambda i, j: (0, 0)),
            pl.BlockSpec((1, 4 * H), lambda i, j: (0, 0)),
            pl.BlockSpec((4 * H, 4 * H), lambda i, j: (0, 0)),
            pl.BlockSpec((1, 4 * H), lambda i, j: (0, 0)),
            pl.BlockSpec((4 * H, 4 * O), lambda i, j: (0, 0)),
            pl.BlockSpec((1, 4 * O), lambda i, j: (0, 0)),
        ],
        out_specs=pl.BlockSpec((B_TILE, KA_PER * 8, 8 * O),
                               lambda i, j: (i, j, 0)),
        out_shape=jax.ShapeDtypeStruct((B, K // 4, 8 * O), jnp.float32),
        scratch_shapes=[pltpu.VMEM((B_TILE, 16, 256), jnp.bfloat16)],
        compiler_params=pltpu.CompilerParams(
            dimension_semantics=("parallel", "arbitrary")),
        interpret=False,
    )(phi, x0a, x0b, x1, w1phiT, e0lo, e0hi, e12, dcat2, w2Tq, b2q, w3Tq, b3q)
    return out.reshape(B, K, 2 * O)
